# Initial kernel scaffold; baseline (speedup 1.0000x reference)
import jax
import jax.numpy as jnp
from jax.experimental import pallas as pl


def kernel(x, edge_index, edge_attr, global_features, batch, aW1, ab1, aW2, ab2, eW1, eb1, eW2, eb2, nW1, nb1, nW2, nb2, lng, lnb, gW, gb, oW1, ob1, oW2, ob2, oW3, ob3):
    raise NotImplementedError("write your pallas kernel here")


# speedup vs baseline: 2.1495x; 2.1495x over previous
import functools

import jax
import jax.numpy as jnp
from jax import lax
from jax.experimental import pallas as pl
from jax.experimental.pallas import tpu as pltpu, tpu_sc as plsc

N = 10000
E = 320000
B = 16
AF = 108
EF = 41
H = 128
L = 4
NT = 5
GF = 239


_NBLK = 10
_BN = N // _NBLK


def _silu(v):
    return v * jax.nn.sigmoid(v)


def _embed_body(x_ref, w1_ref, b1_ref, w2_ref, b2_ref, o_ref):
    t = _silu(jnp.dot(x_ref[...], w1_ref[...], preferred_element_type=jnp.float32)
              + b1_ref[...])
    o_ref[...] = (jnp.dot(t, w2_ref[...], preferred_element_type=jnp.float32)
                  + b2_ref[...])


def _embed(x, aW1, ab1, aW2, ab2):
    return pl.pallas_call(
        _embed_body,
        grid=(_NBLK,),
        in_specs=[
            pl.BlockSpec((_BN, AF), lambda i: (i, 0)),
            pl.BlockSpec((AF, H), lambda i: (0, 0)),
            pl.BlockSpec((1, H), lambda i: (0, 0)),
            pl.BlockSpec((H, H), lambda i: (0, 0)),
            pl.BlockSpec((1, H), lambda i: (0, 0)),
        ],
        out_specs=pl.BlockSpec((_BN, H), lambda i: (i, 0)),
        out_shape=jax.ShapeDtypeStruct((N, H), jnp.float32),
    )(x, aW1, ab1.reshape(1, H), aW2, ab2.reshape(1, H))


_BE = 2000
_NEB = E // _BE


def _edge_body(ea_ref, w1_ref, b1_ref, w2_ref, b2_ref, o_ref):
    t = _silu(jnp.dot(ea_ref[...], w1_ref[0], preferred_element_type=jnp.float32)
              + b1_ref[0])
    o_ref[0] = (jnp.dot(t, w2_ref[0], preferred_element_type=jnp.float32)
                + b2_ref[0])


def _edge_filters(edge_attr, eW1, eb1, eW2, eb2):
    return pl.pallas_call(
        _edge_body,
        grid=(_NEB, L),
        in_specs=[
            pl.BlockSpec((_BE, EF), lambda i, l: (i, 0)),
            pl.BlockSpec((1, EF, H), lambda i, l: (l, 0, 0)),
            pl.BlockSpec((1, 1, H), lambda i, l: (l, 0, 0)),
            pl.BlockSpec((1, H, H), lambda i, l: (l, 0, 0)),
            pl.BlockSpec((1, 1, H), lambda i, l: (l, 0, 0)),
        ],
        out_specs=pl.BlockSpec((1, _BE, H), lambda i, l: (l, i, 0)),
        out_shape=jax.ShapeDtypeStruct((L, E, H), jnp.float32),
    )(edge_attr, eW1, eb1.reshape(L, 1, H), eW2, eb2.reshape(L, 1, H))


_NSC = 2
_NTILE = 16
_NW = _NSC * _NTILE
_EPW = E // _NW
_C = 80
_NCHUNK = _EPW // _C
_NPAD = 10240
_RPT = _NPAD // _NTILE
_ZR = 128


def _make_scatter(layer):
    mesh = plsc.VectorSubcoreMesh(core_axis_name="c", subcore_axis_name="s")

    @functools.partial(
        pl.kernel,
        out_type=jax.ShapeDtypeStruct((_NSC, _NPAD, H), jnp.float32),
        mesh=mesh,
        scratch_types=[
            pltpu.VMEM((_C,), jnp.int32),
            pltpu.VMEM((_C,), jnp.int32),
            pltpu.VMEM((_C, H), jnp.float32),
            pltpu.VMEM((_C, H), jnp.float32),
            pltpu.VMEM((_C, H), jnp.float32),
            pltpu.VMEM((_ZR, H), jnp.float32),
            pltpu.VMEM_SHARED((_NPAD, H), jnp.float32),
            pltpu.SemaphoreType.DMA,
        ],
    )
    def scatter(h_hbm, wall_hbm, src_hbm, dst_hbm, out_hbm,
                src_v, dst_v, rows_v, w_v, msg_v, zero_v, agg_sh, sem):
        c = lax.axis_index("c")
        s = lax.axis_index("s")
        wid = c * _NTILE + s

        def zb(e, _):
            for f in range(H // 16):
                zero_v[e, pl.ds(f * 16, 16)] = jnp.zeros((16,), jnp.float32)
            return 0
        lax.fori_loop(0, _ZR, zb, 0)
        for k in range(_RPT // _ZR):
            pltpu.sync_copy(zero_v, agg_sh.at[pl.ds(s * _RPT + k * _ZR, _ZR)])
        plsc.subcore_barrier()

        def chunk(i, _):
            base = wid * _EPW + i * _C
            pltpu.sync_copy(src_hbm.at[pl.ds(base, _C)], src_v)
            pltpu.sync_copy(dst_hbm.at[pl.ds(base, _C)], dst_v)
            pltpu.async_copy(h_hbm.at[src_v], rows_v, sem).wait()
            pltpu.sync_copy(wall_hbm.at[layer, pl.ds(base, _C)], w_v)

            def mul(e, _):
                for f in range(H // 16):
                    sl = pl.ds(f * 16, 16)
                    msg_v[e, sl] = rows_v[e, sl] * w_v[e, sl]
                return 0
            lax.fori_loop(0, _C, mul, 0)

            pltpu.sync_copy(msg_v, agg_sh.at[dst_v], add=True)
            return 0
        lax.fori_loop(0, _NCHUNK, chunk, 0)

        plsc.subcore_barrier()
        pltpu.sync_copy(agg_sh.at[pl.ds(s * _RPT, _RPT)],
                        out_hbm.at[c, pl.ds(s * _RPT, _RPT)])

    return scatter


def _node_body(p_ref, h_ref, w1_ref, b1_ref, w2_ref, b2_ref, g_ref, bt_ref, o_ref):
    agg = p_ref[0] + p_ref[1]
    t = _silu(jnp.dot(agg, w1_ref[...], preferred_element_type=jnp.float32)
              + b1_ref[...])
    out = jnp.dot(t, w2_ref[...], preferred_element_type=jnp.float32) + b2_ref[...]
    z = h_ref[...] + out
    mu = jnp.mean(z, axis=-1, keepdims=True)
    zc = z - mu
    var = jnp.mean(zc * zc, axis=-1, keepdims=True)
    o_ref[...] = zc / jnp.sqrt(var + 1e-5) * g_ref[...] + bt_ref[...]


def _node(partials, h, nW1l, nb1l, nW2l, nb2l, lngl, lnbl):
    return pl.pallas_call(
        _node_body,
        grid=(_NBLK,),
        in_specs=[
            pl.BlockSpec((_NSC, _BN, H), lambda i: (0, i, 0)),
            pl.BlockSpec((_BN, H), lambda i: (i, 0)),
            pl.BlockSpec((H, H), lambda i: (0, 0)),
            pl.BlockSpec((1, H), lambda i: (0, 0)),
            pl.BlockSpec((H, H), lambda i: (0, 0)),
            pl.BlockSpec((1, H), lambda i: (0, 0)),
            pl.BlockSpec((1, H), lambda i: (0, 0)),
            pl.BlockSpec((1, H), lambda i: (0, 0)),
        ],
        out_specs=pl.BlockSpec((_BN, H), lambda i: (i, 0)),
        out_shape=jax.ShapeDtypeStruct((N, H), jnp.float32),
    )(partials, h, nW1l, nb1l.reshape(1, H), nW2l, nb2l.reshape(1, H),
      lngl.reshape(1, H), lnbl.reshape(1, H))


def _pool_body(h_ref, b_ref, gf_ref, gW_ref, gb_ref, oW1a_ref, oW1b_ref,
               ob1_ref, oW2_ref, ob2_ref, oW3_ref, ob3_ref, o_ref,
               sums_ref, cnts_ref):
    i = pl.program_id(0)

    @pl.when(i == 0)
    def _init():
        sums_ref[...] = jnp.zeros((B, H), jnp.float32)
        cnts_ref[...] = jnp.zeros((B, H), jnp.float32)

    bvec = b_ref[...]
    oh = (bvec == lax.broadcasted_iota(jnp.int32, (_BN, B), 1)).astype(jnp.float32)
    sums_ref[...] += lax.dot_general(oh, h_ref[...], (((0,), (0,)), ((), ())),
                                     preferred_element_type=jnp.float32)
    cnts_ref[...] += lax.dot_general(oh, jnp.ones((_BN, H), jnp.float32),
                                     (((0,), (0,)), ((), ())),
                                     preferred_element_type=jnp.float32)

    @pl.when(i == _NBLK - 1)
    def _head():
        repr_ = sums_ref[...] / jnp.maximum(cnts_ref[...], 1.0)
        gp = _silu(jnp.dot(gf_ref[...], gW_ref[...],
                           preferred_element_type=jnp.float32) + gb_ref[...])
        h1 = _silu(jnp.dot(repr_, oW1a_ref[...], preferred_element_type=jnp.float32)
                   + jnp.dot(gp, oW1b_ref[...], preferred_element_type=jnp.float32)
                   + ob1_ref[...])
        h2 = _silu(jnp.dot(h1, oW2_ref[...], preferred_element_type=jnp.float32)
                   + ob2_ref[...])
        o_ref[...] = (jnp.dot(h2, oW3_ref[...], preferred_element_type=jnp.float32)
                      + ob3_ref[...])


def _pool_head(h, batch2d, gf, gW, gb, oW1, ob1, oW2, ob2, oW3, ob3):
    full = lambda shp: pl.BlockSpec(shp, lambda i: tuple(0 for _ in shp))
    return pl.pallas_call(
        _pool_body,
        grid=(_NBLK,),
        in_specs=[
            pl.BlockSpec((_BN, H), lambda i: (i, 0)),
            pl.BlockSpec((_BN, 1), lambda i: (i, 0)),
            full((B, GF)),
            full((GF, H)),
            full((1, H)),
            full((H, H)),
            full((H, H)),
            full((1, H)),
            full((H, H // 2)),
            full((1, H // 2)),
            full((H // 2, NT)),
            full((1, NT)),
        ],
        out_specs=full((B, NT)),
        out_shape=jax.ShapeDtypeStruct((B, NT), jnp.float32),
        scratch_shapes=[
            pltpu.VMEM((B, H), jnp.float32),
            pltpu.VMEM((B, H), jnp.float32),
        ],
    )(h, batch2d, gf, gW, gb.reshape(1, H), oW1[:H], oW1[H:],
      ob1.reshape(1, H), oW2, ob2.reshape(1, H // 2), oW3, ob3.reshape(1, NT))


def kernel(x, edge_index, edge_attr, global_features, batch,
           aW1, ab1, aW2, ab2,
           eW1, eb1, eW2, eb2, nW1, nb1, nW2, nb2, lng, lnb,
           gW, gb, oW1, ob1, oW2, ob2, oW3, ob3):
    src = edge_index[0]
    dst = edge_index[1]
    h = _embed(x, aW1, ab1, aW2, ab2)
    wall = _edge_filters(edge_attr, eW1, eb1, eW2, eb2)
    for l in range(L):
        partials = _make_scatter(l)(h, wall, src, dst)
        h = _node(partials, h, nW1[l], nb1[l], nW2[l], nb2[l], lng[l], lnb[l])
    return _pool_head(h, batch.reshape(N, 1), global_features, gW, gb,
                      oW1, ob1, oW2, ob2, oW3, ob3)
